# baseline (speedup 1.0000x reference)
import jax
import jax.numpy as jnp
from jax.experimental import pallas as pl
from jax.experimental.pallas import tpu as pltpu

_TS = 512


def _add_pos_kernel(x_ref, pos_ref, o_ref):
    o_ref[...] = x_ref[...] + pos_ref[...][None, :, :]


def kernel(x, pos_table):
    B, S, D = x.shape
    ts = _TS if S % _TS == 0 else S
    grid = (S // ts, B)
    out = pl.pallas_call(
        _add_pos_kernel,
        grid=grid,
        in_specs=[
            pl.BlockSpec((1, ts, D), lambda i, b: (b, i, 0)),
            pl.BlockSpec((ts, D), lambda i, b: (i, 0)),
        ],
        out_specs=pl.BlockSpec((1, ts, D), lambda i, b: (b, i, 0)),
        out_shape=jax.ShapeDtypeStruct((B, S, D), x.dtype),
        compiler_params=pltpu.CompilerParams(
            dimension_semantics=("arbitrary", "arbitrary"),
        ),
    )(x, pos_table)
    return out


# speedup vs baseline: 1.0433x; 1.0433x over previous
import jax
import jax.numpy as jnp
from jax.experimental import pallas as pl
from jax.experimental.pallas import tpu as pltpu

_CH = 512
_NBUF = 5


def _make_body(B, S, D, ch, nbuf):
    st = S // ch
    T = st * B

    def body(x_hbm, pos_hbm, o_hbm, xbuf, posbuf, obuf, in_sems, pos_sems,
             out_sems):
        def chunk(t):
            return divmod(t, B)

        def make_in(t):
            s, b = chunk(t)
            return pltpu.make_async_copy(
                x_hbm.at[b, pl.ds(s * ch, ch), :], xbuf.at[t % nbuf],
                in_sems.at[t % nbuf])

        def make_pos(s):
            return pltpu.make_async_copy(
                pos_hbm.at[pl.ds(s * ch, ch), :], posbuf.at[s % 2],
                pos_sems.at[s % 2])

        def make_out(t):
            s, b = chunk(t)
            return pltpu.make_async_copy(
                obuf.at[t % nbuf], o_hbm.at[b, pl.ds(s * ch, ch), :],
                out_sems.at[t % nbuf])

        in_copies, out_copies, pos_copies = {}, {}, {}
        pos_copies[0] = make_pos(0)
        pos_copies[0].start()
        for t in range(min(nbuf, T)):
            in_copies[t] = make_in(t)
            in_copies[t].start()
        for t in range(T):
            s, b = chunk(t)
            if b == 0:
                pos_copies[s].wait()
                if s + 1 < st:
                    pos_copies[s + 1] = make_pos(s + 1)
                    pos_copies[s + 1].start()
            in_copies[t].wait()
            if t >= nbuf:
                out_copies[t - nbuf].wait()
            obuf[t % nbuf] = xbuf[t % nbuf] + posbuf[s % 2]
            out_copies[t] = make_out(t)
            out_copies[t].start()
            if t + nbuf < T:
                in_copies[t + nbuf] = make_in(t + nbuf)
                in_copies[t + nbuf].start()
        for t in range(max(0, T - nbuf), T):
            out_copies[t].wait()

    return body


def kernel(x, pos_table):
    B, S, D = x.shape
    ch = _CH if S % _CH == 0 else S
    nbuf = min(_NBUF, (S // ch) * B)
    out = pl.pallas_call(
        _make_body(B, S, D, ch, nbuf),
        in_specs=[
            pl.BlockSpec(memory_space=pltpu.HBM),
            pl.BlockSpec(memory_space=pltpu.HBM),
        ],
        out_specs=pl.BlockSpec(memory_space=pltpu.HBM),
        out_shape=jax.ShapeDtypeStruct((B, S, D), x.dtype),
        scratch_shapes=[
            pltpu.VMEM((nbuf, ch, D), x.dtype),
            pltpu.VMEM((2, ch, D), x.dtype),
            pltpu.VMEM((nbuf, ch, D), x.dtype),
            pltpu.SemaphoreType.DMA((nbuf,)),
            pltpu.SemaphoreType.DMA((2,)),
            pltpu.SemaphoreType.DMA((nbuf,)),
        ],
    )(x, pos_table)
    return out
